# baseline (speedup 1.0000x reference)
import functools

import jax
import jax.numpy as jnp
from jax import lax
from jax.experimental import pallas as pl
from jax.experimental.pallas import tpu as pltpu
from jax.experimental.pallas import tpu_sc as plsc

N = 10000
D = 128
E = 320000
NC = 2
NS = 16
NW = NC * NS
B = 128
K = 80
EW = E // NW
EWP = K * B
NBUF = 2
NP = 10240
RPT = NP // NS

_MESH = plsc.VectorSubcoreMesh(
    core_axis_name="c", subcore_axis_name="s", num_cores=NC, num_subcores=NS
)


@functools.partial(
    pl.kernel,
    out_type=jax.ShapeDtypeStruct((NC, NP, D), jnp.float32),
    mesh=_MESH,
    scratch_types=[
        pltpu.VMEM((NBUF, B), jnp.int32),
        pltpu.VMEM((NBUF, B), jnp.int32),
        [pltpu.VMEM((B, D), jnp.float32)] * NBUF,
        pltpu.VMEM_SHARED((NP, D), jnp.float32),
        [pltpu.SemaphoreType.DMA] * NBUF,
        [pltpu.SemaphoreType.DMA] * NBUF,
    ],
)
def _sc_aggregate(x_hbm, col_hbm, row_hbm, zero_hbm, out_hbm,
                  colb, rowb, rbufs, acc, gsems, isems):
    cid = lax.axis_index("c")
    sid = lax.axis_index("s")
    wid = sid * NC + cid
    base = wid * EWP

    def load_idx_sync(j, b):
        pltpu.sync_copy(col_hbm.at[pl.ds(base + j * B, B)], colb.at[b])
        pltpu.sync_copy(row_hbm.at[pl.ds(base + j * B, B)], rowb.at[b])

    def load_idx(j, b):
        pltpu.async_copy(col_hbm.at[pl.ds(base + j * B, B)], colb.at[b],
                         isems[b])
        pltpu.async_copy(row_hbm.at[pl.ds(base + j * B, B)], rowb.at[b],
                         isems[b])

    def wait_idx(b):
        pltpu.make_async_copy(col_hbm.at[pl.ds(base, B)], colb.at[b],
                              isems[b]).wait()
        pltpu.make_async_copy(row_hbm.at[pl.ds(base, B)], rowb.at[b],
                              isems[b]).wait()

    def gather(b):
        pltpu.async_copy(x_hbm.at[colb.at[b]], rbufs[b], gsems[b])

    def wait_gather(b):
        pltpu.make_async_copy(x_hbm.at[colb.at[b]], rbufs[b],
                              gsems[b]).wait()

    def scatter(b):
        pltpu.sync_copy(rbufs[b], acc.at[rowb.at[b]], add=True)

    pltpu.sync_copy(zero_hbm.at[pl.ds(sid * RPT, RPT)],
                    acc.at[pl.ds(sid * RPT, RPT)])
    load_idx_sync(0, 0)
    load_idx_sync(1, 1)
    plsc.subcore_barrier()
    gather(0)

    def body(j, b):
        b1 = 1 - b
        wait_gather(b)

        @pl.when(j > 0)
        def _():
            wait_idx(b1)

        gather(b1)
        scatter(b)

        @pl.when(j < K - 2)
        def _():
            load_idx(j + 2, b)

    def pair(jj, carry):
        body(2 * jj, 0)
        body(2 * jj + 1, 1)
        return carry

    lax.fori_loop(0, (K - 2) // 2, pair, 0)
    wait_gather(0)
    wait_idx(1)
    gather(1)
    scatter(0)
    wait_gather(1)
    scatter(1)
    plsc.subcore_barrier()

    pltpu.sync_copy(acc.at[pl.ds(sid * RPT, RPT)],
                    out_hbm.at[cid, pl.ds(sid * RPT, RPT)])


def _mm_body(p_ref, w_ref, o_ref):
    s = p_ref[0] + p_ref[1]
    o_ref[...] = jnp.maximum(
        jnp.dot(s, w_ref[...], preferred_element_type=jnp.float32), 0.0)


_BM = 1000


def _tc_combine_matmul(p, w):
    return pl.pallas_call(
        _mm_body,
        grid=(N // _BM,),
        in_specs=[
            pl.BlockSpec((NC, _BM, D), lambda i: (0, i, 0)),
            pl.BlockSpec((D, D), lambda i: (0, 0)),
        ],
        out_specs=pl.BlockSpec((_BM, D), lambda i: (i, 0)),
        out_shape=jax.ShapeDtypeStruct((N, D), jnp.float32),
    )(p, w)


def _pad_edges(edge_index):
    npad = EWP - EW
    pad_col = jnp.zeros((NW, npad), jnp.int32)
    pad_row = jnp.broadcast_to(
        N + (jnp.arange(npad, dtype=jnp.int32) % (NP - N)), (NW, npad))
    col = jnp.concatenate([edge_index[1].reshape(NW, EW), pad_col], axis=1)
    row = jnp.concatenate([edge_index[0].reshape(NW, EW), pad_row], axis=1)
    return col.reshape(NW * EWP), row.reshape(NW * EWP)


def kernel(x, edge_index0, edge_index1, W0, W1):
    col0, row0 = _pad_edges(edge_index0)
    col1, row1 = _pad_edges(edge_index1)
    zero = jnp.zeros((NP, D), jnp.float32)

    p0 = _sc_aggregate(x, col0, row0, zero)
    h1 = _tc_combine_matmul(p0, W0)
    p1 = _sc_aggregate(h1, col1, row1, zero)
    return _tc_combine_matmul(p1, W1)
